# baseline (speedup 1.0000x reference)
import functools

import jax
import jax.numpy as jnp
from jax import lax
from jax.experimental import pallas as pl
from jax.experimental.pallas import tpu as pltpu
from jax.experimental.pallas import tpu_sc as plsc

B = 16384
F = 8
V = 12
D = 64
D2 = 2 * D
ROWS = B * F
TAB = F * V

NC = 2
NS = 16
NW = NC * NS
PER_W = ROWS // NW
CHUNK = 128
NCHUNK = PER_W // CHUNK
LANES = 16
GROUPS = CHUNK // LANES
NBUF = 4


def _sc_gather_body(x_hbm, tab_hbm, out_hbm, idx_v, tab_v, *rest):
    bufs = rest[:NBUF]
    psems = rest[NBUF:2 * NBUF]

    wid = lax.axis_index("s") * NC + lax.axis_index("c")
    base = wid * PER_W

    pltpu.sync_copy(tab_hbm, tab_v)
    pltpu.sync_copy(x_hbm.at[wid], idx_v)

    iota = lax.iota(jnp.int32, LANES)
    offv = (iota % F) * V

    def do_chunk(t, bi):
        buf = bufs[bi]

        def group(g, _):
            iv = idx_v[pl.ds(t * CHUNK + g * LANES, LANES)]
            rowbase = (iv + offv) * D2
            outbase = (g * LANES + iota) * D2
            for c in range(D2):
                val = plsc.load_gather(tab_v, [rowbase + c])
                plsc.store_scatter(buf, [outbase + c], val)
            return 0

        lax.fori_loop(0, GROUPS, group, 0)
        return pltpu.async_copy(
            buf,
            out_hbm.at[pl.ds((base + t * CHUNK) * D2, CHUNK * D2)],
            psems[bi])

    def drain(b):
        pltpu.make_async_copy(
            bufs[b], out_hbm.at[pl.ds(base * D2, CHUNK * D2)],
            psems[b]).wait()

    for b in range(NBUF):
        do_chunk(b, b)

    def outer(gi, _):
        for b in range(NBUF):
            drain(b)
            do_chunk(gi * NBUF + b, b)
        return 0

    lax.fori_loop(1, NCHUNK // NBUF, outer, 0)
    for b in range(NBUF):
        drain(b)


_sc_gather = functools.partial(
    pl.kernel,
    out_type=jax.ShapeDtypeStruct((ROWS * D2,), jnp.float32),
    mesh=plsc.VectorSubcoreMesh(core_axis_name="c", subcore_axis_name="s"),
    compiler_params=pltpu.CompilerParams(needs_layout_passes=False),
    scratch_types=(
        [pltpu.VMEM((PER_W,), jnp.int32),
         pltpu.VMEM((TAB * D2,), jnp.float32)]
        + [pltpu.VMEM((CHUNK * D2,), jnp.float32) for _ in range(NBUF)]
        + [pltpu.SemaphoreType.DMA for _ in range(NBUF)]
    ),
)(_sc_gather_body)


def kernel(x, mean_table, std_table):
    tab = jnp.concatenate(
        [mean_table.reshape(TAB, D), std_table.reshape(TAB, D)],
        axis=1).reshape(TAB * D2)
    x2 = x.reshape(NW, PER_W).astype(jnp.int32)
    out = _sc_gather(x2, tab)
    return out.reshape(B, F, D2)


# speedup vs baseline: 1.4409x; 1.4409x over previous
import functools

import jax
import jax.numpy as jnp
from jax import lax
from jax.experimental import pallas as pl
from jax.experimental.pallas import tpu as pltpu
from jax.experimental.pallas import tpu_sc as plsc

B = 16384
F = 8
V = 12
D = 64
D2 = 2 * D
ROWS = B * F
TAB = F * V

NC = 2
NS = 16
NW = NC * NS
PER_W = ROWS // NW
CHUNK = 256
NCHUNK = PER_W // CHUNK
LANES = 16
GROUPS = CHUNK // LANES
NBUF = 2
BLK = 8


def _sc_gather_body(x_hbm, tab_hbm, out_hbm, idx_v, tab_v, *rest):
    bufs = rest[:NBUF]
    psems = rest[NBUF:2 * NBUF]

    wid = lax.axis_index("s") * NC + lax.axis_index("c")
    base = wid * PER_W

    pltpu.sync_copy(tab_hbm, tab_v)
    pltpu.sync_copy(x_hbm.at[wid], idx_v)

    iota = lax.iota(jnp.int32, LANES)
    offv = (iota % F) * V

    def do_chunk(t, bi):
        buf = bufs[bi]

        def group(g, _):
            iv = idx_v[pl.ds(t * CHUNK + g * LANES, LANES)]
            rowbase = (iv + offv) * D2
            outbase = (g * LANES + iota) * D2
            for w in range(D2 // BLK):
                cols = [w * BLK + j for j in range(BLK)]
                vals = [plsc.load_gather(tab_v, [rowbase + c]) for c in cols]
                for c, v in zip(cols, vals):
                    plsc.store_scatter(buf, [outbase + c], v)
            return 0

        lax.fori_loop(0, GROUPS, group, 0)
        return pltpu.async_copy(
            buf,
            out_hbm.at[pl.ds((base + t * CHUNK) * D2, CHUNK * D2)],
            psems[bi])

    def drain(b):
        pltpu.make_async_copy(
            bufs[b], out_hbm.at[pl.ds(base * D2, CHUNK * D2)],
            psems[b]).wait()

    for b in range(NBUF):
        do_chunk(b, b)

    def outer(gi, _):
        for b in range(NBUF):
            drain(b)
            do_chunk(gi * NBUF + b, b)
        return 0

    lax.fori_loop(1, NCHUNK // NBUF, outer, 0)
    for b in range(NBUF):
        drain(b)


_sc_gather = functools.partial(
    pl.kernel,
    out_type=jax.ShapeDtypeStruct((ROWS * D2,), jnp.float32),
    mesh=plsc.VectorSubcoreMesh(core_axis_name="c", subcore_axis_name="s"),
    compiler_params=pltpu.CompilerParams(needs_layout_passes=False),
    scratch_types=(
        [pltpu.VMEM((PER_W,), jnp.int32),
         pltpu.VMEM((TAB * D2,), jnp.float32)]
        + [pltpu.VMEM((CHUNK * D2,), jnp.float32) for _ in range(NBUF)]
        + [pltpu.SemaphoreType.DMA for _ in range(NBUF)]
    ),
)(_sc_gather_body)


def kernel(x, mean_table, std_table):
    tab = jnp.concatenate(
        [mean_table.reshape(TAB, D), std_table.reshape(TAB, D)],
        axis=1).reshape(TAB * D2)
    x2 = x.reshape(NW, PER_W).astype(jnp.int32)
    out = _sc_gather(x2, tab)
    return out.reshape(B, F, D2)


# speedup vs baseline: 10.5210x; 7.3015x over previous
import functools

import jax
import jax.numpy as jnp
from jax import lax
from jax.experimental import pallas as pl
from jax.experimental.pallas import tpu as pltpu
from jax.experimental.pallas import tpu_sc as plsc

B = 16384
F = 8
V = 12
D = 64
D2 = 2 * D
ROWS = B * F
TAB = F * V

NC = 2
NS = 16
NW = NC * NS
PER_W = ROWS // NW
CHUNK = 128
NCHUNK = PER_W // CHUNK
LANES = 16
NBUF = 6
LA = 3


def _sc_gather_body(x_hbm, tab_hbm, out_hbm, idx_v, tab_stage, tab_sp, *rest):
    pairs = rest[:NBUF // 2]
    gsems = rest[NBUF // 2:NBUF // 2 + NBUF]
    psems = rest[NBUF // 2 + NBUF:]
    bufs = [pairs[i // 2].at[pl.ds((i % 2) * CHUNK, CHUNK)]
            for i in range(NBUF)]

    sid = lax.axis_index("s")
    wid = sid * NC + lax.axis_index("c")
    base = wid * PER_W

    @pl.when(sid == 0)
    def _stage_table():
        pltpu.sync_copy(tab_hbm, tab_stage)
        pltpu.sync_copy(tab_stage, tab_sp)

    pltpu.sync_copy(x_hbm.at[wid], idx_v)
    off = (lax.iota(jnp.int32, 16) % F) * V

    plsc.subcore_barrier()

    gat = [None] * NBUF
    put = [None] * (NBUF // 2)
    for t in range(NCHUNK + LA):
        if t < NCHUNK:
            bi = t % NBUF
            if put[bi // 2] is not None and bi % 2 == 0:
                put[bi // 2].wait()
            for o in range(CHUNK // LANES):
                sl = pl.ds(o * LANES, LANES)
                idx_v[t, sl] = idx_v[t, sl] + off
            gat[bi] = pltpu.async_copy(
                tab_sp.at[idx_v.at[t]], bufs[bi], gsems[bi])
        if t >= LA and (t - LA) % 2 == 1:
            c = t - LA - 1
            pb = c % NBUF
            gat[pb].wait()
            gat[pb + 1].wait()
            put[pb // 2] = pltpu.async_copy(
                pairs[pb // 2],
                out_hbm.at[pl.ds(base + c * CHUNK, 2 * CHUNK)],
                psems[pb // 2])
    for p in put:
        if p is not None:
            p.wait()


_sc_gather = functools.partial(
    pl.kernel,
    out_type=jax.ShapeDtypeStruct((ROWS, D2), jnp.float32),
    mesh=plsc.VectorSubcoreMesh(core_axis_name="c", subcore_axis_name="s"),
    scratch_types=(
        [pltpu.VMEM((NCHUNK, CHUNK), jnp.int32),
         pltpu.VMEM((TAB, D2), jnp.float32),
         pltpu.VMEM_SHARED((TAB, D2), jnp.float32)]
        + [pltpu.VMEM((2 * CHUNK, D2), jnp.float32) for _ in range(NBUF // 2)]
        + [pltpu.SemaphoreType.DMA for _ in range(NBUF + NBUF // 2)]
    ),
)(_sc_gather_body)


def kernel(x, mean_table, std_table):
    tab = jnp.concatenate(
        [mean_table.reshape(TAB, D), std_table.reshape(TAB, D)], axis=1)
    x3 = x.reshape(NW, NCHUNK, CHUNK).astype(jnp.int32)
    out = _sc_gather(x3, tab)
    return out.reshape(B, F, D2)
